# baseline (speedup 1.0000x reference)
import functools

import jax
import jax.numpy as jnp
from jax import lax
from jax.experimental import pallas as pl
from jax.experimental.pallas import tpu as pltpu
from jax.experimental.pallas import tpu_sc as plsc

N = 10000
E = 320000
D = 128
L = 16
GROUP = 128
NC = 2
NS = 16
NW = NC * NS
NGT = 80
G_PAD = NW * NGT
E_PAD = G_PAD * GROUP
NB = 2
CH = 8
NCH = NGT // CH
PCHUNK = 80
NPC = N // PCHUNK
PC_TILE = NPC // NS
PC_REM = NPC - PC_TILE * NS
MM_BLK = 2000


def _sc_layer(x, comb, wgt):
    mesh = plsc.VectorSubcoreMesh(core_axis_name="c", subcore_axis_name="s")

    @functools.partial(
        pl.kernel,
        out_type=jax.ShapeDtypeStruct((NC, N, D), jnp.float32),
        mesh=mesh,
        scratch_types=[
            pltpu.VMEM_SHARED((N, D), jnp.float32),
            pltpu.VMEM((CH, 2, GROUP), jnp.int32),
            pltpu.VMEM((CH, 2, GROUP), jnp.int32),
            pltpu.VMEM((CH, GROUP), jnp.float32),
            pltpu.VMEM((CH, GROUP), jnp.float32),
            pltpu.VMEM((NB * GROUP, D), jnp.float32),
            pltpu.SemaphoreType.DMA((NB,)),
            pltpu.SemaphoreType.DMA,
        ],
    )
    def sc_kernel(x_hbm, comb_hbm, w_hbm, out_hbm,
                  acc, cb0, cb1, wb0, wb1, rows, gsem, psem):
        c = lax.axis_index("c")
        s = lax.axis_index("s")
        wid = c * NS + s
        g0 = wid * NGT

        cp0 = pltpu.async_copy(comb_hbm.at[pl.ds(g0, CH)], cb0, psem)
        cw0 = pltpu.async_copy(w_hbm.at[pl.ds(g0, CH)], wb0, psem)

        def _zero(r, carry):
            for j in range(D // L):
                rows[r, pl.ds(j * L, L)] = jnp.zeros((L,), jnp.float32)
            return carry
        lax.fori_loop(0, PCHUNK, _zero, 0)
        pc0 = s * PC_TILE + jnp.minimum(s, PC_REM)
        pcnt = PC_TILE + jnp.where(s < PC_REM, 1, 0)

        def _zacc(k, carry):
            pltpu.sync_copy(rows.at[pl.ds(0, PCHUNK)],
                            acc.at[pl.ds((pc0 + k) * PCHUNK, PCHUNK)])
            return carry
        lax.fori_loop(0, pcnt, _zacc, 0)
        cp0.wait()
        cw0.wait()
        plsc.subcore_barrier()

        def slot(b):
            return rows.at[pl.ds(b * GROUP, GROUP)]

        def wait_gather(b):
            pltpu.make_async_copy(
                x_hbm.at[pl.ds(0, GROUP)], slot(b), gsem.at[b]).wait()

        def process(cb_cur, wb_cur, lgl, b):
            wait_gather(b)

            def scale16(eb, carry):
                wv16 = wb_cur[lgl, pl.ds(eb * L, L)]
                for lane in range(L):
                    wv = jnp.full((L,), wv16[lane], jnp.float32)
                    e = b * GROUP + eb * L + lane
                    for j in range(D // L):
                        rows[e, pl.ds(j * L, L)] = (
                            rows[e, pl.ds(j * L, L)] * wv)
                return carry
            lax.fori_loop(0, GROUP // L, scale16, 0)
            pltpu.sync_copy(slot(b), acc.at[cb_cur.at[lgl, 1]], add=True)

        def gather(cb, lgl, b):
            pltpu.async_copy(x_hbm.at[cb.at[lgl, 0]], slot(b), gsem.at[b])

        for b in range(NB):
            gather(cb0, b, b)

        @pl.loop(0, NCH, step=2)
        def _chunks(k):
            for half, (cb_cur, cb_nxt, wb_cur, wb_nxt) in enumerate(
                    ((cb0, cb1, wb0, wb1), (cb1, cb0, wb1, wb0))):
                q = k + half

                @pl.when(q + 1 < NCH)
                def _pref():
                    pltpu.async_copy(
                        comb_hbm.at[pl.ds(g0 + (q + 1) * CH, CH)],
                        cb_nxt, psem)
                    pltpu.async_copy(
                        w_hbm.at[pl.ds(g0 + (q + 1) * CH, CH)],
                        wb_nxt, psem)

                @pl.loop(0, CH - NB, step=NB)
                def _inner(kk):
                    for b in range(NB):
                        process(cb_cur, wb_cur, kk + b, b)
                        gather(cb_cur, kk + b + NB, b)

                @pl.when(q + 1 < NCH)
                def _wait_pref():
                    pltpu.make_async_copy(
                        comb_hbm.at[pl.ds(0, CH)], cb_nxt, psem).wait()
                    pltpu.make_async_copy(
                        w_hbm.at[pl.ds(0, CH)], wb_nxt, psem).wait()
                for b in range(NB):
                    process(cb_cur, wb_cur, CH - NB + b, b)

                    @pl.when(q + 1 < NCH)
                    def _refill():
                        gather(cb_nxt, b, b)

        plsc.subcore_barrier()

        def _pub(k, carry):
            r0 = (pc0 + k) * PCHUNK
            pltpu.sync_copy(acc.at[pl.ds(r0, PCHUNK)],
                            rows.at[pl.ds(0, PCHUNK)])
            pltpu.sync_copy(rows.at[pl.ds(0, PCHUNK)],
                            out_hbm.at[c, pl.ds(r0, PCHUNK)])
            return carry
        lax.fori_loop(0, pcnt, _pub, 0)

    return sc_kernel(x, comb, wgt)


def _mm_fused(p, b, W):
    def body(p_ref, b_ref, w_ref, o_ref):
        hs = p_ref[0] + p_ref[1]
        o_ref[...] = jnp.dot(hs, w_ref[...],
                             preferred_element_type=jnp.float32) + b_ref[...]
    return pl.pallas_call(
        body,
        grid=(N // MM_BLK,),
        in_specs=[pl.BlockSpec((NC, MM_BLK, D), lambda i: (0, i, 0)),
                  pl.BlockSpec((1, D), lambda i: (0, 0)),
                  pl.BlockSpec((D, D), lambda i: (0, 0))],
        out_specs=pl.BlockSpec((MM_BLK, D), lambda i: (i, 0)),
        out_shape=jax.ShapeDtypeStruct((N, D), jnp.float32),
    )(p, b, W)


def kernel(x, edge_index, edge_weight, W1, b1, W2, b2):
    pad = E_PAD - E
    src = jnp.concatenate(
        [edge_index[0], jnp.zeros((pad,), jnp.int32)]).reshape(G_PAD, GROUP)
    dst = jnp.concatenate(
        [edge_index[1], jnp.zeros((pad,), jnp.int32)]).reshape(G_PAD, GROUP)
    wgt = jnp.concatenate(
        [edge_weight, jnp.zeros((pad,), jnp.float32)]).reshape(G_PAD, GROUP)
    comb = jnp.stack([src, dst], axis=1)
    b1r = b1.reshape(1, D)
    b2r = b2.reshape(1, D)

    p1 = _sc_layer(x, comb, wgt)
    h1 = _mm_fused(p1, b1r, W1)
    p2 = _sc_layer(h1, comb, wgt)
    return _mm_fused(p2, b2r, W2)
